# baseline (speedup 1.0000x reference)
import functools

import jax
import jax.numpy as jnp
import numpy as np
from jax.experimental import pallas as pl
from jax.experimental.pallas import tpu as pltpu

_PAR = pltpu.CompilerParams(dimension_semantics=("parallel",))

H, W = 64, 128
B = 2
N_TRACED = 39
N_CONST = 5
HIDDEN = 32
HEADS = 4
HH = HIDDEN * HEADS
EDGE_DIM = 8
NON_LOCAL = 128
INNER_DIM = 3 * N_TRACED
R = B * H * W
BH = B * H
G = 32
NT = BH // G
RT = G * W
TPB = H // G
NEG = -1e30


def _lrelu(x, s):
    return jnp.where(x >= 0, x, s * x)


def _cm(c):
    return pl.BlockSpec((c, RT), lambda m: (0, m))


def _cm_prev(c):
    return pl.BlockSpec((c, W), lambda m: (0, jnp.maximum(G * m - 1, 0)))


def _cm_next(c):
    return pl.BlockSpec((c, W), lambda m: (0, jnp.minimum(G * m + G, BH - 1)))


def _full(a, b):
    return pl.BlockSpec((a, b), lambda m: (0, 0))


def _lane_iota(shape):
    return jax.lax.broadcasted_iota(jnp.int32, shape, 1)


def _gat_kernel(nparts, *refs):
    xs = refs[:3 * nparts]
    eTc, eTp, eTn = refs[3 * nparts:3 * nparts + 3]
    wts = refs[3 * nparts + 3:4 * nparts + 3]
    mblkT = refs[4 * nparts + 3]
    asrcT = refs[4 * nparts + 4]
    adstT = refs[4 * nparts + 5]
    biasT = refs[4 * nparts + 6]
    out = refs[4 * nparts + 7]

    def hpart(sel):
        h = jnp.dot(wts[0][...], xs[sel][...].astype(jnp.bfloat16),
                    preferred_element_type=jnp.float32)
        for i in range(1, nparts):
            h = h + jnp.dot(wts[i][...], xs[3 * i + sel][...].astype(
                jnp.bfloat16), preferred_element_type=jnp.float32)
        return h

    h_c = hpart(0)
    h_p = hpart(1)
    h_n = hpart(2)
    he = jnp.concatenate([h_p, h_c, h_n], axis=1)
    a_src = asrcT[...] @ he
    a_dst = adstT[...] @ h_c
    ee = jnp.concatenate([eTp[...], eTc[...], eTn[...]], axis=1)
    a_e = mblkT[...] @ ee

    m = pl.program_id(0)
    NE = RT + 2 * W

    def shifts(x):
        nr = x.shape[0]
        ji = _lane_iota((nr, RT)) % W
        d0 = x[:, 2 * W:]
        d3 = x[:, :RT]
        b1 = x[:, W + 1:NE - W + 1]
        f1 = x[:, 1:RT + 1]
        d1 = jnp.where(ji == W - 1, f1, b1)
        b2 = x[:, W - 1:NE - W - 1]
        f2 = x[:, 2 * W - 1:NE - 1]
        d2 = jnp.where(ji == 0, f2, b2)
        return d0, d1, d2, d3

    s_sh = shifts(a_src)
    e_sh = shifts(a_e)
    gi = (m * G + _lane_iota((HEADS, RT)) // W) % H
    masks = [gi < H - 1, None, None, gi > 0]
    alphas = []
    for d in range(4):
        al = s_sh[d] + a_dst + e_sh[d][4 * d:4 * d + 4]
        al = _lrelu(al, 0.2)
        if masks[d] is not None:
            al = jnp.where(masks[d], al, NEG)
        alphas.append(al)
    amax = jnp.maximum(jnp.maximum(alphas[0], alphas[1]),
                       jnp.maximum(alphas[2], alphas[3]))
    exs = [jnp.exp(a - amax) for a in alphas]
    den = exs[0] + exs[1] + exs[2] + exs[3] + 1e-16

    expandT = (jax.lax.broadcasted_iota(jnp.int32, (HH, HEADS), 0) // HIDDEN
               == jax.lax.broadcasted_iota(jnp.int32, (HH, HEADS), 1)
               ).astype(jnp.float32)

    h_sh = shifts(he)
    acc = jnp.zeros((HH, RT), jnp.float32)
    for d in range(4):
        wfull = expandT @ (exs[d] / den)
        acc = acc + h_sh[d] * wfull
    out[...] = _lrelu(acc + biasT[...], 0.01)


def _gat_layer(x_parts, e32T, lp):
    nparts = len(x_parts)
    c0 = x_parts[0].shape[0]
    w_parts = ([lp['W'].T] if nparts == 1 else
               [lp['W'][:c0].T, lp['W'][c0:].T])
    in_specs = []
    args = []
    for p in x_parts:
        c = p.shape[0]
        in_specs += [_cm(c), _cm_prev(c), _cm_next(c)]
        args += [p, p, p]
    in_specs += [_cm(4 * EDGE_DIM), _cm_prev(4 * EDGE_DIM),
                 _cm_next(4 * EDGE_DIM)]
    args += [e32T, e32T, e32T]
    for w in w_parts:
        in_specs.append(_full(w.shape[0], w.shape[1]))
        args.append(w.astype(jnp.bfloat16))
    in_specs += [_full(4 * HEADS, 4 * EDGE_DIM), _full(HEADS, HH),
                 _full(HEADS, HH), _full(HH, 1)]
    args += [_edge_mat(lp['W_e'], lp['att_e']).T, _att_mat(lp['att_src']).T,
             _att_mat(lp['att_dst']).T, lp['b'].reshape(HH, 1)]
    return pl.pallas_call(
        functools.partial(_gat_kernel, nparts),
        grid=(NT,),
        compiler_params=_PAR,
        in_specs=in_specs,
        out_specs=_cm(HH),
        out_shape=jax.ShapeDtypeStruct((HH, R), jnp.float32),
    )(*args)


def _conv_kernel(x_ref, xp_ref, xn_ref, k_ref, cb_ref,
                 conv_ref, ps_ref, pss_ref):
    m = pl.program_id(0)

    def roll64(r):
        return jnp.concatenate([r[:, W // 2:], r[:, :W // 2]], axis=1)

    top = m % TPB == 0
    bot = m % TPB == TPB - 1
    left = jnp.where(top, roll64(x_ref[:, W:2 * W]), xp_ref[...])
    right = jnp.where(bot, roll64(x_ref[:, RT - 2 * W:RT - W]), xn_ref[...])
    he = jnp.concatenate([left, x_ref[...], right],
                         axis=1).astype(jnp.bfloat16)
    he = jnp.concatenate([he[:, :W], he, he[:, :W]], axis=1)
    ji = _lane_iota((HH, RT)) % W

    acc = jnp.zeros((NON_LOCAL, RT), jnp.float32)
    for di in range(3):
        for dj in range(3):
            s = W + di * W + dj - 1
            base = he[:, s:s + RT]
            if dj == 0:
                f = he[:, s + W:s + W + RT]
                tap = jnp.where(ji == 0, f, base)
            elif dj == 2:
                f = he[:, s - W:s - W + RT]
                tap = jnp.where(ji == W - 1, f, base)
            else:
                tap = base
            acc = acc + jnp.dot(k_ref[di * 3 + dj], tap,
                                preferred_element_type=jnp.float32)
    acc = acc + cb_ref[...]
    conv_ref[...] = acc
    ps_ref[...] = jnp.sum(acc, axis=1).reshape(1, NON_LOCAL, 1)
    pss_ref[...] = jnp.sum(acc * acc, axis=1).reshape(1, NON_LOCAL, 1)


def _conv(xmT, kmatT, conv_b):
    return pl.pallas_call(
        _conv_kernel,
        grid=(NT,),
        compiler_params=_PAR,
        in_specs=[_cm(HH), _cm_prev(HH), _cm_next(HH),
                  pl.BlockSpec((9, NON_LOCAL, HH), lambda m: (0, 0, 0)),
                  _full(NON_LOCAL, 1)],
        out_specs=[_cm(NON_LOCAL),
                   pl.BlockSpec((1, NON_LOCAL, 1), lambda m: (m, 0, 0)),
                   pl.BlockSpec((1, NON_LOCAL, 1), lambda m: (m, 0, 0))],
        out_shape=[jax.ShapeDtypeStruct((NON_LOCAL, R), jnp.float32),
                   jax.ShapeDtypeStruct((NT, NON_LOCAL, 1), jnp.float32),
                   jax.ShapeDtypeStruct((NT, NON_LOCAL, 1), jnp.float32)],
    )(xmT, xmT, xmT, kmatT.astype(jnp.bfloat16), conv_b.reshape(NON_LOCAL, 1))


def _stats_kernel(ps_ref, pss_ref, g_ref, b_ref, scale_ref, shift_ref):
    mu = jnp.sum(ps_ref[...].reshape(NT, NON_LOCAL), axis=0) \
        .reshape(NON_LOCAL, 1) / R
    ex2 = jnp.sum(pss_ref[...].reshape(NT, NON_LOCAL), axis=0) \
        .reshape(NON_LOCAL, 1) / R
    var = ex2 - mu * mu
    scale = g_ref[...] * jax.lax.rsqrt(var + 1e-5)
    scale_ref[...] = scale
    shift_ref[...] = b_ref[...] - mu * scale


def _stats(ps, pss, bn_g, bn_b):
    return pl.pallas_call(
        _stats_kernel,
        out_shape=[jax.ShapeDtypeStruct((NON_LOCAL, 1), jnp.float32),
                   jax.ShapeDtypeStruct((NON_LOCAL, 1), jnp.float32)],
    )(ps, pss, bn_g.reshape(NON_LOCAL, 1), bn_b.reshape(NON_LOCAL, 1))


def _mlp_kernel(hx_ref, cv_ref, ps_ref, pss_ref, g_ref, bb_ref, cur_ref,
                w0a_ref, w0b_ref, b0_ref, w1_ref, b1_ref, w2_ref, b2_ref,
                out_ref):
    mu = jnp.sum(ps_ref[...], axis=0) / R
    var = jnp.sum(pss_ref[...], axis=0) / R - mu * mu
    scale = g_ref[...] * jax.lax.rsqrt(var + 1e-5)
    shift = bb_ref[...] - mu * scale
    nl = _lrelu(cv_ref[...] * scale + shift, 0.01)
    z = (jnp.dot(w0a_ref[...], hx_ref[...].astype(jnp.bfloat16),
                 preferred_element_type=jnp.float32)
         + jnp.dot(w0b_ref[...], nl.astype(jnp.bfloat16),
                   preferred_element_type=jnp.float32) + b0_ref[...])
    z = _lrelu(z, 0.01).astype(jnp.bfloat16)
    z = _lrelu(jnp.dot(w1_ref[...], z, preferred_element_type=jnp.float32)
               + b1_ref[...], 0.01).astype(jnp.bfloat16)
    out_ref[...] = (jnp.dot(w2_ref[...], z, preferred_element_type=jnp.float32)
                    + b2_ref[...] + cur_ref[...])


def _mlp(hxT, convT, ps, pss, bn_g, bn_b, curT, w0, b0, w1, b1, w2, b2,
         target):
    return pl.pallas_call(
        _mlp_kernel,
        grid=(NT,),
        compiler_params=_PAR,
        in_specs=[_cm(HH), _cm(NON_LOCAL),
                  pl.BlockSpec((NT, NON_LOCAL, 1), lambda m: (0, 0, 0)),
                  pl.BlockSpec((NT, NON_LOCAL, 1), lambda m: (0, 0, 0)),
                  _full(NON_LOCAL, 1), _full(NON_LOCAL, 1), _cm(target),
                  _full(512, HH), _full(512, NON_LOCAL), _full(512, 1),
                  _full(256, 512), _full(256, 1),
                  _full(target, 256), _full(target, 1)],
        out_specs=_cm(target),
        out_shape=jax.ShapeDtypeStruct((target, R), jnp.float32),
    )(hxT, convT, ps, pss, bn_g.reshape(NON_LOCAL, 1),
      bn_b.reshape(NON_LOCAL, 1), curT,
      w0[:HH].T.astype(jnp.bfloat16), w0[HH:].T.astype(jnp.bfloat16),
      b0.reshape(512, 1), w1.T.astype(jnp.bfloat16), b1.reshape(256, 1),
      w2.T.astype(jnp.bfloat16), b2.reshape(target, 1))


def _att_mat(att):
    out = jnp.zeros((HEADS, HIDDEN, HEADS), jnp.float32)
    for hd in range(HEADS):
        out = out.at[hd, :, hd].set(att[hd])
    return out.reshape(HH, HEADS)


def _edge_mat(w_e, att_e):
    mm = (w_e.reshape(EDGE_DIM, HEADS, HIDDEN) * att_e[None]).sum(-1)
    out = jnp.zeros((4, EDGE_DIM, 4, HEADS), jnp.float32)
    for d in range(4):
        out = out.at[d, :, d, :].set(mm)
    return out.reshape(4 * EDGE_DIM, 4 * HEADS)


def _block(xpT, consT, e32T, p, target):
    h1 = _gat_layer([xpT, consT], e32T, p['gat'][0])
    h2 = _gat_layer([h1], e32T, p['gat'][1])
    kmatT = jnp.transpose(p['conv_w'], (2, 3, 0, 1)) \
        .reshape(9, NON_LOCAL, HH)
    conv, ps, pss = _conv(h2, kmatT, p['conv_b'])
    return _mlp(h2, conv, ps, pss, p['bn_g'], p['bn_b'], xpT[-target:],
                p['mlp_w0'], p['mlp_b0'], p['mlp_w1'], p['mlp_b1'],
                p['mlp_w2'], p['mlp_b2'], target)


def kernel(x, x_cons, time_embedding, y, lat, edge_attr, edge_index, params,
           metric=None):
    te = -jnp.cos(2.0 * np.pi * time_embedding / 8760.0)
    te = jnp.broadcast_to(te.reshape(B, 1, 1), (B, 1, H * W))
    consT = jnp.concatenate([x_cons.reshape(B, N_CONST, H * W), te], axis=1)
    consT = jnp.transpose(consT, (1, 0, 2)).reshape(N_CONST + 1, R)
    xpT = jnp.transpose(x.reshape(B, INNER_DIM, H * W),
                        (1, 0, 2)).reshape(INNER_DIM, R)
    e32T = jnp.transpose(edge_attr.reshape(R, 4 * EDGE_DIM), (1, 0))

    out1 = _block(xpT, consT, e32T, params['blocks'][0], INNER_DIM)
    out2 = _block(out1, consT, e32T, params['blocks'][1], N_TRACED)
    return jnp.transpose(out2.reshape(N_TRACED, B, H * W),
                         (1, 0, 2)).reshape(B, N_TRACED, H, W)


# speedup vs baseline: 1.0349x; 1.0349x over previous
import functools

import jax
import jax.numpy as jnp
import numpy as np
from jax.experimental import pallas as pl
from jax.experimental.pallas import tpu as pltpu

_PAR = pltpu.CompilerParams(dimension_semantics=("parallel",))

H, W = 64, 128
B = 2
N_TRACED = 39
N_CONST = 5
HIDDEN = 32
HEADS = 4
HH = HIDDEN * HEADS
EDGE_DIM = 8
NON_LOCAL = 128
INNER_DIM = 3 * N_TRACED
R = B * H * W
BH = B * H
G = 32
NT = BH // G
RT = G * W
TPB = H // G
NEG = -1e30


def _lrelu(x, s):
    return jnp.where(x >= 0, x, s * x)


def _cm(c):
    return pl.BlockSpec((c, RT), lambda m: (0, m))


def _cm_prev(c):
    return pl.BlockSpec((c, W), lambda m: (0, jnp.maximum(G * m - 1, 0)))


def _cm_next(c):
    return pl.BlockSpec((c, W), lambda m: (0, jnp.minimum(G * m + G, BH - 1)))


def _full(a, b):
    return pl.BlockSpec((a, b), lambda m: (0, 0))


def _lane_iota(shape):
    return jax.lax.broadcasted_iota(jnp.int32, shape, 1)


def _gat_kernel(nparts, *refs):
    xs = refs[:3 * nparts]
    eTc, eTp, eTn = refs[3 * nparts:3 * nparts + 3]
    wts = refs[3 * nparts + 3:4 * nparts + 3]
    mblkT = refs[4 * nparts + 3]
    asrcT = refs[4 * nparts + 4]
    adstT = refs[4 * nparts + 5]
    biasT = refs[4 * nparts + 6]
    out = refs[4 * nparts + 7]

    def hpart(sel):
        h = jnp.dot(wts[0][...], xs[sel][...].astype(jnp.bfloat16),
                    preferred_element_type=jnp.float32)
        for i in range(1, nparts):
            h = h + jnp.dot(wts[i][...], xs[3 * i + sel][...].astype(
                jnp.bfloat16), preferred_element_type=jnp.float32)
        return h


    h_c = hpart(0)
    h_p = hpart(1)
    h_n = hpart(2)
    he = jnp.concatenate([h_p, h_c, h_n], axis=1)
    a_src = asrcT[...] @ he
    a_dst = adstT[...] @ h_c
    ee = jnp.concatenate([eTp[...], eTc[...], eTn[...]], axis=1)
    a_e = mblkT[...] @ ee

    m = pl.program_id(0)
    NE = RT + 2 * W

    def shifts(x):
        nr = x.shape[0]
        ji = _lane_iota((nr, RT)) % W
        d0 = x[:, 2 * W:]
        d3 = x[:, :RT]
        b1 = x[:, W + 1:NE - W + 1]
        f1 = x[:, 1:RT + 1]
        d1 = jnp.where(ji == W - 1, f1, b1)
        b2 = x[:, W - 1:NE - W - 1]
        f2 = x[:, 2 * W - 1:NE - 1]
        d2 = jnp.where(ji == 0, f2, b2)
        return d0, d1, d2, d3

    s_sh = shifts(a_src)
    e_sh = shifts(a_e)
    gi = (m * G + _lane_iota((HEADS, RT)) // W) % H
    masks = [gi < H - 1, None, None, gi > 0]
    alphas = []
    for d in range(4):
        al = s_sh[d] + a_dst + e_sh[d][4 * d:4 * d + 4]
        al = _lrelu(al, 0.2)
        if masks[d] is not None:
            al = jnp.where(masks[d], al, NEG)
        alphas.append(al)
    amax = jnp.maximum(jnp.maximum(alphas[0], alphas[1]),
                       jnp.maximum(alphas[2], alphas[3]))
    exs = [jnp.exp(a - amax) for a in alphas]
    den = exs[0] + exs[1] + exs[2] + exs[3] + 1e-16

    h_sh = shifts(he)
    heads_acc = []
    for hd in range(HEADS):
        acc = jnp.zeros((HIDDEN, RT), jnp.float32)
        for d in range(4):
            wrow = (exs[d][hd:hd + 1] / den[hd:hd + 1])
            acc = acc + h_sh[d][HIDDEN * hd:HIDDEN * (hd + 1)] * wrow
        heads_acc.append(acc)
    res = jnp.concatenate(heads_acc, axis=0) + biasT[...]
    out[...] = _lrelu(res, 0.01).astype(jnp.bfloat16)


def _gat_layer(x_parts, e32T, lp):
    nparts = len(x_parts)
    c0 = x_parts[0].shape[0]
    w_parts = ([lp['W'].T] if nparts == 1 else
               [lp['W'][:c0].T, lp['W'][c0:].T])
    in_specs = []
    args = []
    for p in x_parts:
        c = p.shape[0]
        in_specs += [_cm(c), _cm_prev(c), _cm_next(c)]
        args += [p, p, p]
    in_specs += [_cm(4 * EDGE_DIM), _cm_prev(4 * EDGE_DIM),
                 _cm_next(4 * EDGE_DIM)]
    args += [e32T, e32T, e32T]
    for w in w_parts:
        in_specs.append(_full(w.shape[0], w.shape[1]))
        args.append(w.astype(jnp.bfloat16))
    in_specs += [_full(4 * HEADS, 4 * EDGE_DIM), _full(HEADS, HH),
                 _full(HEADS, HH), _full(HH, 1)]
    args += [_edge_mat(lp['W_e'], lp['att_e']).T, _att_mat(lp['att_src']).T,
             _att_mat(lp['att_dst']).T, lp['b'].reshape(HH, 1)]
    return pl.pallas_call(
        functools.partial(_gat_kernel, nparts),
        grid=(NT,),
        compiler_params=_PAR,
        in_specs=in_specs,
        out_specs=_cm(HH),
        out_shape=jax.ShapeDtypeStruct((HH, R), jnp.bfloat16),
    )(*args)


def _conv_kernel(x_ref, xp_ref, xn_ref, k_ref, cb_ref,
                 conv_ref, ps_ref, pss_ref):
    m = pl.program_id(0)

    def roll64(r):
        return jnp.concatenate([r[:, W // 2:], r[:, :W // 2]], axis=1)

    top = m % TPB == 0
    bot = m % TPB == TPB - 1
    left = jnp.where(top, roll64(x_ref[:, W:2 * W]), xp_ref[...])
    right = jnp.where(bot, roll64(x_ref[:, RT - 2 * W:RT - W]), xn_ref[...])
    he = jnp.concatenate([left, x_ref[...], right], axis=1)
    he = jnp.concatenate([he[:, :W], he, he[:, :W]], axis=1)
    ji = _lane_iota((HH, RT)) % W

    acc = jnp.zeros((NON_LOCAL, RT), jnp.float32)
    for di in range(3):
        for dj in range(3):
            s = W + di * W + dj - 1
            base = he[:, s:s + RT]
            if dj == 0:
                f = he[:, s + W:s + W + RT]
                tap = jnp.where(ji == 0, f, base)
            elif dj == 2:
                f = he[:, s - W:s - W + RT]
                tap = jnp.where(ji == W - 1, f, base)
            else:
                tap = base
            acc = acc + jnp.dot(k_ref[di * 3 + dj], tap,
                                preferred_element_type=jnp.float32)
    acc = acc + cb_ref[...]
    conv_ref[...] = acc.astype(jnp.bfloat16)
    ps_ref[...] = jnp.sum(acc, axis=1).reshape(1, NON_LOCAL, 1)
    pss_ref[...] = jnp.sum(acc * acc, axis=1).reshape(1, NON_LOCAL, 1)


def _conv(xmT, kmatT, conv_b):
    return pl.pallas_call(
        _conv_kernel,
        grid=(NT,),
        compiler_params=_PAR,
        in_specs=[_cm(HH), _cm_prev(HH), _cm_next(HH),
                  pl.BlockSpec((9, NON_LOCAL, HH), lambda m: (0, 0, 0)),
                  _full(NON_LOCAL, 1)],
        out_specs=[_cm(NON_LOCAL),
                   pl.BlockSpec((1, NON_LOCAL, 1), lambda m: (m, 0, 0)),
                   pl.BlockSpec((1, NON_LOCAL, 1), lambda m: (m, 0, 0))],
        out_shape=[jax.ShapeDtypeStruct((NON_LOCAL, R), jnp.bfloat16),
                   jax.ShapeDtypeStruct((NT, NON_LOCAL, 1), jnp.float32),
                   jax.ShapeDtypeStruct((NT, NON_LOCAL, 1), jnp.float32)],
    )(xmT, xmT, xmT, kmatT.astype(jnp.bfloat16), conv_b.reshape(NON_LOCAL, 1))


def _stats_kernel(ps_ref, pss_ref, g_ref, b_ref, scale_ref, shift_ref):
    mu = jnp.sum(ps_ref[...].reshape(NT, NON_LOCAL), axis=0) \
        .reshape(NON_LOCAL, 1) / R
    ex2 = jnp.sum(pss_ref[...].reshape(NT, NON_LOCAL), axis=0) \
        .reshape(NON_LOCAL, 1) / R
    var = ex2 - mu * mu
    scale = g_ref[...] * jax.lax.rsqrt(var + 1e-5)
    scale_ref[...] = scale
    shift_ref[...] = b_ref[...] - mu * scale


def _stats(ps, pss, bn_g, bn_b):
    return pl.pallas_call(
        _stats_kernel,
        out_shape=[jax.ShapeDtypeStruct((NON_LOCAL, 1), jnp.float32),
                   jax.ShapeDtypeStruct((NON_LOCAL, 1), jnp.float32)],
    )(ps, pss, bn_g.reshape(NON_LOCAL, 1), bn_b.reshape(NON_LOCAL, 1))


def _mlp_kernel(hx_ref, cv_ref, ps_ref, pss_ref, g_ref, bb_ref, cur_ref,
                w0a_ref, w0b_ref, b0_ref, w1_ref, b1_ref, w2_ref, b2_ref,
                out_ref):
    mu = jnp.sum(ps_ref[...], axis=0) / R
    var = jnp.sum(pss_ref[...], axis=0) / R - mu * mu
    scale = g_ref[...] * jax.lax.rsqrt(var + 1e-5)
    shift = bb_ref[...] - mu * scale
    nl = _lrelu(cv_ref[...].astype(jnp.float32) * scale + shift, 0.01)
    z = (jnp.dot(w0a_ref[...], hx_ref[...],
                 preferred_element_type=jnp.float32)
         + jnp.dot(w0b_ref[...], nl.astype(jnp.bfloat16),
                   preferred_element_type=jnp.float32) + b0_ref[...])
    z = _lrelu(z, 0.01).astype(jnp.bfloat16)
    z = _lrelu(jnp.dot(w1_ref[...], z, preferred_element_type=jnp.float32)
               + b1_ref[...], 0.01).astype(jnp.bfloat16)
    out_ref[...] = (jnp.dot(w2_ref[...], z, preferred_element_type=jnp.float32)
                    + b2_ref[...] + cur_ref[...])


def _mlp(hxT, convT, ps, pss, bn_g, bn_b, curT, w0, b0, w1, b1, w2, b2,
         target):
    return pl.pallas_call(
        _mlp_kernel,
        grid=(NT,),
        compiler_params=_PAR,
        in_specs=[_cm(HH), _cm(NON_LOCAL),
                  pl.BlockSpec((NT, NON_LOCAL, 1), lambda m: (0, 0, 0)),
                  pl.BlockSpec((NT, NON_LOCAL, 1), lambda m: (0, 0, 0)),
                  _full(NON_LOCAL, 1), _full(NON_LOCAL, 1), _cm(target),
                  _full(512, HH), _full(512, NON_LOCAL), _full(512, 1),
                  _full(256, 512), _full(256, 1),
                  _full(target, 256), _full(target, 1)],
        out_specs=_cm(target),
        out_shape=jax.ShapeDtypeStruct((target, R), jnp.float32),
    )(hxT, convT, ps, pss, bn_g.reshape(NON_LOCAL, 1),
      bn_b.reshape(NON_LOCAL, 1), curT,
      w0[:HH].T.astype(jnp.bfloat16), w0[HH:].T.astype(jnp.bfloat16),
      b0.reshape(512, 1), w1.T.astype(jnp.bfloat16), b1.reshape(256, 1),
      w2.T.astype(jnp.bfloat16), b2.reshape(target, 1))


def _att_mat(att):
    out = jnp.zeros((HEADS, HIDDEN, HEADS), jnp.float32)
    for hd in range(HEADS):
        out = out.at[hd, :, hd].set(att[hd])
    return out.reshape(HH, HEADS)


def _edge_mat(w_e, att_e):
    mm = (w_e.reshape(EDGE_DIM, HEADS, HIDDEN) * att_e[None]).sum(-1)
    out = jnp.zeros((4, EDGE_DIM, 4, HEADS), jnp.float32)
    for d in range(4):
        out = out.at[d, :, d, :].set(mm)
    return out.reshape(4 * EDGE_DIM, 4 * HEADS)


def _block(xpT, consT, e32T, p, target):
    h1 = _gat_layer([xpT, consT], e32T, p['gat'][0])
    h2 = _gat_layer([h1], e32T, p['gat'][1])
    kmatT = jnp.transpose(p['conv_w'], (2, 3, 0, 1)) \
        .reshape(9, NON_LOCAL, HH)
    conv, ps, pss = _conv(h2, kmatT, p['conv_b'])
    return _mlp(h2, conv, ps, pss, p['bn_g'], p['bn_b'], xpT[-target:],
                p['mlp_w0'], p['mlp_b0'], p['mlp_w1'], p['mlp_b1'],
                p['mlp_w2'], p['mlp_b2'], target)


def kernel(x, x_cons, time_embedding, y, lat, edge_attr, edge_index, params,
           metric=None):
    te = -jnp.cos(2.0 * np.pi * time_embedding / 8760.0)
    te = jnp.broadcast_to(te.reshape(B, 1, 1), (B, 1, H * W))
    consT = jnp.concatenate([x_cons.reshape(B, N_CONST, H * W), te], axis=1)
    consT = jnp.transpose(consT, (1, 0, 2)).reshape(N_CONST + 1, R)
    xpT = jnp.transpose(x.reshape(B, INNER_DIM, H * W),
                        (1, 0, 2)).reshape(INNER_DIM, R)
    e32T = jnp.transpose(edge_attr.reshape(R, 4 * EDGE_DIM), (1, 0))

    out1 = _block(xpT, consT, e32T, params['blocks'][0], INNER_DIM)
    out2 = _block(out1, consT, e32T, params['blocks'][1], N_TRACED)
    return jnp.transpose(out2.reshape(N_TRACED, B, H * W),
                         (1, 0, 2)).reshape(B, N_TRACED, H, W)


# speedup vs baseline: 1.0666x; 1.0306x over previous
import functools

import jax
import jax.numpy as jnp
import numpy as np
from jax.experimental import pallas as pl
from jax.experimental.pallas import tpu as pltpu

_PAR = pltpu.CompilerParams(dimension_semantics=("parallel",))

H, W = 64, 128
B = 2
N_TRACED = 39
N_CONST = 5
HIDDEN = 32
HEADS = 4
HH = HIDDEN * HEADS
EDGE_DIM = 8
NON_LOCAL = 128
INNER_DIM = 3 * N_TRACED
R = B * H * W
BH = B * H
G = 32
NT = BH // G
RT = G * W
TPB = H // G
NEG = -1e30


def _lrelu(x, s):
    return jnp.where(x >= 0, x, s * x)


def _cm(c):
    return pl.BlockSpec((c, RT), lambda m: (0, m))


def _cm_prev(c):
    return pl.BlockSpec((c, W), lambda m: (0, jnp.maximum(G * m - 1, 0)))


def _cm_next(c):
    return pl.BlockSpec((c, W), lambda m: (0, jnp.minimum(G * m + G, BH - 1)))


def _full(a, b):
    return pl.BlockSpec((a, b), lambda m: (0, 0))


def _lane_iota(shape):
    return jax.lax.broadcasted_iota(jnp.int32, shape, 1)


def _attn(h_ext, ae_ext, asrcT, adstT, biasT, m, ext):
    n = h_ext.shape[1] - 2 * W
    a_src = asrcT[...] @ h_ext
    a_dst = (adstT[...] @ h_ext)[:, W:W + n]
    a_e = ae_ext

    def shifts(x):
        nr = x.shape[0]
        ji = _lane_iota((nr, n)) % W
        d0 = x[:, 2 * W:2 * W + n]
        d3 = x[:, :n]
        d1 = jnp.where(ji == W - 1, x[:, 1:1 + n], x[:, W + 1:W + 1 + n])
        d2 = jnp.where(ji == 0, x[:, 2 * W - 1:2 * W - 1 + n],
                       x[:, W - 1:W - 1 + n])
        return d0, d1, d2, d3

    s_sh = shifts(a_src)
    e_sh = shifts(a_e)
    gi = (m * G + H - ext + 1 + _lane_iota((HEADS, n)) // W) % H
    masks = [gi < H - 1, None, None, gi > 0]
    alphas = []
    for d in range(4):
        al = s_sh[d] + a_dst + e_sh[d][4 * d:4 * d + 4]
        al = _lrelu(al, 0.2)
        if masks[d] is not None:
            al = jnp.where(masks[d], al, NEG)
        alphas.append(al)
    amax = jnp.maximum(jnp.maximum(alphas[0], alphas[1]),
                       jnp.maximum(alphas[2], alphas[3]))
    exs = [jnp.exp(a - amax) for a in alphas]
    den = exs[0] + exs[1] + exs[2] + exs[3] + 1e-16

    h_sh = shifts(h_ext)
    heads_acc = []
    for hd in range(HEADS):
        acc = jnp.zeros((HIDDEN, n), jnp.float32)
        for d in range(4):
            wrow = (exs[d][hd:hd + 1] / den[hd:hd + 1])
            acc = acc + h_sh[d][HIDDEN * hd:HIDDEN * (hd + 1)] * wrow
        heads_acc.append(acc)
    res = jnp.concatenate(heads_acc, axis=0) + biasT[...]
    return _lrelu(res, 0.01)


def _gat_kernel(nparts, *refs):
    k = 5 * nparts
    xs = refs[:k]
    es = refs[k:k + 5]
    wts1 = refs[k + 5:k + 5 + nparts]
    mblk1T, asrc1T, adst1T, bias1T = refs[k + 5 + nparts:k + 9 + nparts]
    w2T, mblk2T, asrc2T, adst2T, bias2T = refs[k + 9 + nparts:k + 14 + nparts]
    out = refs[k + 14 + nparts]

    m = pl.program_id(0)

    def xcat(i):
        rs = refs[5 * i:5 * i + 5]
        return jnp.concatenate(
            [rs[1][...], rs[2][...], rs[0][...], rs[3][...], rs[4][...]],
            axis=1).astype(jnp.bfloat16)

    h1e = jnp.dot(wts1[0][...], xcat(0), preferred_element_type=jnp.float32)
    for i in range(1, nparts):
        h1e = h1e + jnp.dot(wts1[i][...], xcat(i),
                            preferred_element_type=jnp.float32)
    ee = jnp.concatenate(
        [es[1][...], es[2][...], es[0][...], es[3][...], es[4][...]],
        axis=1)
    ae1 = mblk1T[...] @ ee
    g1 = _attn(h1e, ae1, asrc1T, adst1T, bias1T, m, 2)

    h2e = jnp.dot(w2T[...], g1.astype(jnp.bfloat16),
                  preferred_element_type=jnp.float32)
    ae2 = (mblk2T[...] @ ee)[:, W:W + RT + 2 * W]
    g2 = _attn(h2e, ae2, asrc2T, adst2T, bias2T, m, 1)
    out[...] = g2.astype(jnp.bfloat16)


def _cm_off(c, off):
    if off < 0:
        return pl.BlockSpec(
            (c, W), lambda m: (0, jnp.maximum(G * m + off, 0)))
    return pl.BlockSpec(
        (c, W), lambda m: (0, jnp.minimum(G * m + G + off, BH - 1)))


def _gat_block(x_parts, e32T, lps):
    lp1, lp2 = lps
    nparts = len(x_parts)
    c0 = x_parts[0].shape[0]
    w_parts = ([lp1['W'].T] if nparts == 1 else
               [lp1['W'][:c0].T, lp1['W'][c0:].T])
    in_specs = []
    args = []
    for p in x_parts:
        c = p.shape[0]
        in_specs += [_cm(c), _cm_off(c, -2), _cm_off(c, -1),
                     _cm_off(c, 0), _cm_off(c, 1)]
        args += [p] * 5
    in_specs += [_cm(4 * EDGE_DIM), _cm_off(4 * EDGE_DIM, -2),
                 _cm_off(4 * EDGE_DIM, -1), _cm_off(4 * EDGE_DIM, 0),
                 _cm_off(4 * EDGE_DIM, 1)]
    args += [e32T] * 5
    for w in w_parts:
        in_specs.append(_full(w.shape[0], w.shape[1]))
        args.append(w.astype(jnp.bfloat16))
    def lp_args(lp):
        return [_edge_mat(lp['W_e'], lp['att_e']).T,
                _att_mat(lp['att_src']).T, _att_mat(lp['att_dst']).T,
                lp['b'].reshape(HH, 1)]
    in_specs += [_full(4 * HEADS, 4 * EDGE_DIM), _full(HEADS, HH),
                 _full(HEADS, HH), _full(HH, 1)]
    args += lp_args(lp1)
    in_specs += [_full(HH, HH), _full(4 * HEADS, 4 * EDGE_DIM),
                 _full(HEADS, HH), _full(HEADS, HH), _full(HH, 1)]
    args += [lp2['W'].T.astype(jnp.bfloat16)] + lp_args(lp2)
    return pl.pallas_call(
        functools.partial(_gat_kernel, nparts),
        grid=(NT,),
        compiler_params=_PAR,
        in_specs=in_specs,
        out_specs=_cm(HH),
        out_shape=jax.ShapeDtypeStruct((HH, R), jnp.bfloat16),
    )(*args)


def _conv_kernel(x_ref, xp_ref, xn_ref, k_ref, cb_ref,
                 conv_ref, ps_ref, pss_ref):
    m = pl.program_id(0)

    def roll64(r):
        return jnp.concatenate([r[:, W // 2:], r[:, :W // 2]], axis=1)

    top = m % TPB == 0
    bot = m % TPB == TPB - 1
    left = jnp.where(top, roll64(x_ref[:, W:2 * W]), xp_ref[...])
    right = jnp.where(bot, roll64(x_ref[:, RT - 2 * W:RT - W]), xn_ref[...])
    he = jnp.concatenate([left, x_ref[...], right], axis=1)
    he = jnp.concatenate([he[:, :W], he, he[:, :W]], axis=1)
    ji = _lane_iota((HH, RT)) % W

    acc = jnp.zeros((NON_LOCAL, RT), jnp.float32)
    for di in range(3):
        for dj in range(3):
            s = W + di * W + dj - 1
            base = he[:, s:s + RT]
            if dj == 0:
                f = he[:, s + W:s + W + RT]
                tap = jnp.where(ji == 0, f, base)
            elif dj == 2:
                f = he[:, s - W:s - W + RT]
                tap = jnp.where(ji == W - 1, f, base)
            else:
                tap = base
            acc = acc + jnp.dot(k_ref[di * 3 + dj], tap,
                                preferred_element_type=jnp.float32)
    acc = acc + cb_ref[...]
    conv_ref[...] = acc.astype(jnp.bfloat16)
    ps_ref[...] = jnp.sum(acc, axis=1).reshape(1, NON_LOCAL, 1)
    pss_ref[...] = jnp.sum(acc * acc, axis=1).reshape(1, NON_LOCAL, 1)


def _conv(xmT, kmatT, conv_b):
    return pl.pallas_call(
        _conv_kernel,
        grid=(NT,),
        compiler_params=_PAR,
        in_specs=[_cm(HH), _cm_prev(HH), _cm_next(HH),
                  pl.BlockSpec((9, NON_LOCAL, HH), lambda m: (0, 0, 0)),
                  _full(NON_LOCAL, 1)],
        out_specs=[_cm(NON_LOCAL),
                   pl.BlockSpec((1, NON_LOCAL, 1), lambda m: (m, 0, 0)),
                   pl.BlockSpec((1, NON_LOCAL, 1), lambda m: (m, 0, 0))],
        out_shape=[jax.ShapeDtypeStruct((NON_LOCAL, R), jnp.bfloat16),
                   jax.ShapeDtypeStruct((NT, NON_LOCAL, 1), jnp.float32),
                   jax.ShapeDtypeStruct((NT, NON_LOCAL, 1), jnp.float32)],
    )(xmT, xmT, xmT, kmatT.astype(jnp.bfloat16), conv_b.reshape(NON_LOCAL, 1))


def _stats_kernel(ps_ref, pss_ref, g_ref, b_ref, scale_ref, shift_ref):
    mu = jnp.sum(ps_ref[...].reshape(NT, NON_LOCAL), axis=0) \
        .reshape(NON_LOCAL, 1) / R
    ex2 = jnp.sum(pss_ref[...].reshape(NT, NON_LOCAL), axis=0) \
        .reshape(NON_LOCAL, 1) / R
    var = ex2 - mu * mu
    scale = g_ref[...] * jax.lax.rsqrt(var + 1e-5)
    scale_ref[...] = scale
    shift_ref[...] = b_ref[...] - mu * scale


def _stats(ps, pss, bn_g, bn_b):
    return pl.pallas_call(
        _stats_kernel,
        out_shape=[jax.ShapeDtypeStruct((NON_LOCAL, 1), jnp.float32),
                   jax.ShapeDtypeStruct((NON_LOCAL, 1), jnp.float32)],
    )(ps, pss, bn_g.reshape(NON_LOCAL, 1), bn_b.reshape(NON_LOCAL, 1))


def _mlp_kernel(hx_ref, cv_ref, ps_ref, pss_ref, g_ref, bb_ref, cur_ref,
                w0a_ref, w0b_ref, b0_ref, w1_ref, b1_ref, w2_ref, b2_ref,
                out_ref):
    mu = jnp.sum(ps_ref[...], axis=0) / R
    var = jnp.sum(pss_ref[...], axis=0) / R - mu * mu
    scale = g_ref[...] * jax.lax.rsqrt(var + 1e-5)
    shift = bb_ref[...] - mu * scale
    nl = _lrelu(cv_ref[...].astype(jnp.float32) * scale + shift, 0.01)
    z = (jnp.dot(w0a_ref[...], hx_ref[...],
                 preferred_element_type=jnp.float32)
         + jnp.dot(w0b_ref[...], nl.astype(jnp.bfloat16),
                   preferred_element_type=jnp.float32) + b0_ref[...])
    z = _lrelu(z, 0.01).astype(jnp.bfloat16)
    z = _lrelu(jnp.dot(w1_ref[...], z, preferred_element_type=jnp.float32)
               + b1_ref[...], 0.01).astype(jnp.bfloat16)
    out_ref[...] = (jnp.dot(w2_ref[...], z, preferred_element_type=jnp.float32)
                    + b2_ref[...] + cur_ref[...])


def _mlp(hxT, convT, ps, pss, bn_g, bn_b, curT, w0, b0, w1, b1, w2, b2,
         target):
    return pl.pallas_call(
        _mlp_kernel,
        grid=(NT,),
        compiler_params=_PAR,
        in_specs=[_cm(HH), _cm(NON_LOCAL),
                  pl.BlockSpec((NT, NON_LOCAL, 1), lambda m: (0, 0, 0)),
                  pl.BlockSpec((NT, NON_LOCAL, 1), lambda m: (0, 0, 0)),
                  _full(NON_LOCAL, 1), _full(NON_LOCAL, 1), _cm(target),
                  _full(512, HH), _full(512, NON_LOCAL), _full(512, 1),
                  _full(256, 512), _full(256, 1),
                  _full(target, 256), _full(target, 1)],
        out_specs=_cm(target),
        out_shape=jax.ShapeDtypeStruct((target, R), jnp.float32),
    )(hxT, convT, ps, pss, bn_g.reshape(NON_LOCAL, 1),
      bn_b.reshape(NON_LOCAL, 1), curT,
      w0[:HH].T.astype(jnp.bfloat16), w0[HH:].T.astype(jnp.bfloat16),
      b0.reshape(512, 1), w1.T.astype(jnp.bfloat16), b1.reshape(256, 1),
      w2.T.astype(jnp.bfloat16), b2.reshape(target, 1))


def _att_mat(att):
    out = jnp.zeros((HEADS, HIDDEN, HEADS), jnp.float32)
    for hd in range(HEADS):
        out = out.at[hd, :, hd].set(att[hd])
    return out.reshape(HH, HEADS)


def _edge_mat(w_e, att_e):
    mm = (w_e.reshape(EDGE_DIM, HEADS, HIDDEN) * att_e[None]).sum(-1)
    out = jnp.zeros((4, EDGE_DIM, 4, HEADS), jnp.float32)
    for d in range(4):
        out = out.at[d, :, d, :].set(mm)
    return out.reshape(4 * EDGE_DIM, 4 * HEADS)


def _block(xpT, consT, e32T, p, target):
    h2 = _gat_block([xpT, consT], e32T, p['gat'])
    kmatT = jnp.transpose(p['conv_w'], (2, 3, 0, 1)) \
        .reshape(9, NON_LOCAL, HH)
    conv, ps, pss = _conv(h2, kmatT, p['conv_b'])
    return _mlp(h2, conv, ps, pss, p['bn_g'], p['bn_b'], xpT[-target:],
                p['mlp_w0'], p['mlp_b0'], p['mlp_w1'], p['mlp_b1'],
                p['mlp_w2'], p['mlp_b2'], target)


def kernel(x, x_cons, time_embedding, y, lat, edge_attr, edge_index, params,
           metric=None):
    te = -jnp.cos(2.0 * np.pi * time_embedding / 8760.0)
    te = jnp.broadcast_to(te.reshape(B, 1, 1), (B, 1, H * W))
    consT = jnp.concatenate([x_cons.reshape(B, N_CONST, H * W), te], axis=1)
    consT = jnp.transpose(consT, (1, 0, 2)).reshape(N_CONST + 1, R)
    xpT = jnp.transpose(x.reshape(B, INNER_DIM, H * W),
                        (1, 0, 2)).reshape(INNER_DIM, R)
    e32T = jnp.transpose(edge_attr.reshape(R, 4 * EDGE_DIM), (1, 0))

    out1 = _block(xpT, consT, e32T, params['blocks'][0], INNER_DIM)
    out2 = _block(out1, consT, e32T, params['blocks'][1], N_TRACED)
    return jnp.transpose(out2.reshape(N_TRACED, B, H * W),
                         (1, 0, 2)).reshape(B, N_TRACED, H, W)


# speedup vs baseline: 1.0901x; 1.0221x over previous
import functools

import jax
import jax.numpy as jnp
import numpy as np
from jax.experimental import pallas as pl
from jax.experimental.pallas import tpu as pltpu

_PAR = pltpu.CompilerParams(dimension_semantics=("parallel",))

H, W = 64, 128
B = 2
N_TRACED = 39
N_CONST = 5
HIDDEN = 32
HEADS = 4
HH = HIDDEN * HEADS
EDGE_DIM = 8
NON_LOCAL = 128
INNER_DIM = 3 * N_TRACED
R = B * H * W
BH = B * H
G = 32
NT = BH // G
RT = G * W
TPB = H // G
NEG = -1e30


def _lrelu(x, s):
    return jnp.where(x >= 0, x, s * x)


def _cm(c):
    return pl.BlockSpec((c, RT), lambda m: (0, m))


def _cm_prev(c):
    return pl.BlockSpec((c, W), lambda m: (0, jnp.maximum(G * m - 1, 0)))


def _cm_next(c):
    return pl.BlockSpec((c, W), lambda m: (0, jnp.minimum(G * m + G, BH - 1)))


def _full(a, b):
    return pl.BlockSpec((a, b), lambda m: (0, 0))


def _lane_iota(shape):
    return jax.lax.broadcasted_iota(jnp.int32, shape, 1)


def _attn(h_ext, ae_ext, asrcT, adstT, biasT, m, ext):
    n = h_ext.shape[1] - 2 * W
    a_src = asrcT[...] @ h_ext
    a_dst = (adstT[...] @ h_ext)[:, W:W + n]
    a_e = ae_ext

    def shifts(x):
        nr = x.shape[0]
        ji = _lane_iota((nr, n)) % W
        d0 = x[:, 2 * W:2 * W + n]
        d3 = x[:, :n]
        d1 = jnp.where(ji == W - 1, x[:, 1:1 + n], x[:, W + 1:W + 1 + n])
        d2 = jnp.where(ji == 0, x[:, 2 * W - 1:2 * W - 1 + n],
                       x[:, W - 1:W - 1 + n])
        return d0, d1, d2, d3

    s_sh = shifts(a_src)
    e_sh = shifts(a_e)
    gi = (m * G + H - ext + 1 + _lane_iota((HEADS, n)) // W) % H
    masks = [gi < H - 1, None, None, gi > 0]
    alphas = []
    for d in range(4):
        al = s_sh[d] + a_dst + e_sh[d][4 * d:4 * d + 4]
        al = _lrelu(al, 0.2)
        if masks[d] is not None:
            al = jnp.where(masks[d], al, NEG)
        alphas.append(al)
    amax = jnp.maximum(jnp.maximum(alphas[0], alphas[1]),
                       jnp.maximum(alphas[2], alphas[3]))
    exs = [jnp.exp(a - amax) for a in alphas]
    den = exs[0] + exs[1] + exs[2] + exs[3] + 1e-16

    h_sh = shifts(h_ext.astype(jnp.bfloat16))
    ws = [(exs[d] / den).astype(jnp.bfloat16) for d in range(4)]
    heads_acc = []
    for hd in range(HEADS):
        acc = jnp.zeros((HIDDEN, n), jnp.bfloat16)
        for d in range(4):
            acc = acc + h_sh[d][HIDDEN * hd:HIDDEN * (hd + 1)] \
                * ws[d][hd:hd + 1]
        heads_acc.append(acc)
    res = jnp.concatenate(heads_acc, axis=0) + biasT[...]
    return _lrelu(res, 0.01).astype(jnp.bfloat16)


def _gat_kernel(nparts, *refs):
    k = 5 * nparts
    xs = refs[:k]
    es = refs[k:k + 5]
    wts1 = refs[k + 5:k + 5 + nparts]
    mblk1T, asrc1T, adst1T, bias1T = refs[k + 5 + nparts:k + 9 + nparts]
    w2T, mblk2T, asrc2T, adst2T, bias2T = refs[k + 9 + nparts:k + 14 + nparts]
    out = refs[k + 14 + nparts]

    m = pl.program_id(0)

    def xcat(i):
        rs = refs[5 * i:5 * i + 5]
        return jnp.concatenate(
            [rs[1][...], rs[2][...], rs[0][...], rs[3][...], rs[4][...]],
            axis=1).astype(jnp.bfloat16)

    h1e = jnp.dot(wts1[0][...], xcat(0), preferred_element_type=jnp.float32)
    for i in range(1, nparts):
        h1e = h1e + jnp.dot(wts1[i][...], xcat(i),
                            preferred_element_type=jnp.float32)
    ee = jnp.concatenate(
        [es[1][...], es[2][...], es[0][...], es[3][...], es[4][...]],
        axis=1)
    ae1 = mblk1T[...] @ ee
    g1 = _attn(h1e, ae1, asrc1T, adst1T, bias1T, m, 2)

    h2e = jnp.dot(w2T[...], g1, preferred_element_type=jnp.float32)
    ae2 = (mblk2T[...] @ ee)[:, W:W + RT + 2 * W]
    out[...] = _attn(h2e, ae2, asrc2T, adst2T, bias2T, m, 1)


def _cm_off(c, off):
    if off < 0:
        return pl.BlockSpec(
            (c, W), lambda m: (0, jnp.maximum(G * m + off, 0)))
    return pl.BlockSpec(
        (c, W), lambda m: (0, jnp.minimum(G * m + G + off, BH - 1)))


def _gat_block(x_parts, e32T, lps):
    lp1, lp2 = lps
    nparts = len(x_parts)
    c0 = x_parts[0].shape[0]
    w_parts = ([lp1['W'].T] if nparts == 1 else
               [lp1['W'][:c0].T, lp1['W'][c0:].T])
    in_specs = []
    args = []
    for p in x_parts:
        c = p.shape[0]
        in_specs += [_cm(c), _cm_off(c, -2), _cm_off(c, -1),
                     _cm_off(c, 0), _cm_off(c, 1)]
        args += [p] * 5
    in_specs += [_cm(4 * EDGE_DIM), _cm_off(4 * EDGE_DIM, -2),
                 _cm_off(4 * EDGE_DIM, -1), _cm_off(4 * EDGE_DIM, 0),
                 _cm_off(4 * EDGE_DIM, 1)]
    args += [e32T] * 5
    for w in w_parts:
        in_specs.append(_full(w.shape[0], w.shape[1]))
        args.append(w.astype(jnp.bfloat16))
    def lp_args(lp):
        return [_edge_mat(lp['W_e'], lp['att_e']).T,
                _att_mat(lp['att_src']).T, _att_mat(lp['att_dst']).T,
                lp['b'].reshape(HH, 1)]
    in_specs += [_full(4 * HEADS, 4 * EDGE_DIM), _full(HEADS, HH),
                 _full(HEADS, HH), _full(HH, 1)]
    args += lp_args(lp1)
    in_specs += [_full(HH, HH), _full(4 * HEADS, 4 * EDGE_DIM),
                 _full(HEADS, HH), _full(HEADS, HH), _full(HH, 1)]
    args += [lp2['W'].T.astype(jnp.bfloat16)] + lp_args(lp2)
    return pl.pallas_call(
        functools.partial(_gat_kernel, nparts),
        grid=(NT,),
        compiler_params=_PAR,
        in_specs=in_specs,
        out_specs=_cm(HH),
        out_shape=jax.ShapeDtypeStruct((HH, R), jnp.bfloat16),
    )(*args)


def _conv_kernel(x_ref, xp_ref, xn_ref, k_ref, cb_ref,
                 conv_ref, ps_ref, pss_ref):
    m = pl.program_id(0)

    def roll64(r):
        return jnp.concatenate([r[:, W // 2:], r[:, :W // 2]], axis=1)

    top = m % TPB == 0
    bot = m % TPB == TPB - 1
    left = jnp.where(top, roll64(x_ref[:, W:2 * W]), xp_ref[...])
    right = jnp.where(bot, roll64(x_ref[:, RT - 2 * W:RT - W]), xn_ref[...])
    he = jnp.concatenate([left, x_ref[...], right], axis=1)
    he = jnp.concatenate([he[:, :W], he, he[:, :W]], axis=1)
    ji = _lane_iota((HH, RT)) % W

    acc = jnp.zeros((NON_LOCAL, RT), jnp.float32)
    for di in range(3):
        for dj in range(3):
            s = W + di * W + dj - 1
            base = he[:, s:s + RT]
            if dj == 0:
                f = he[:, s + W:s + W + RT]
                tap = jnp.where(ji == 0, f, base)
            elif dj == 2:
                f = he[:, s - W:s - W + RT]
                tap = jnp.where(ji == W - 1, f, base)
            else:
                tap = base
            acc = acc + jnp.dot(k_ref[di * 3 + dj], tap,
                                preferred_element_type=jnp.float32)
    acc = acc + cb_ref[...]
    conv_ref[...] = acc.astype(jnp.bfloat16)
    ps_ref[...] = jnp.sum(acc, axis=1).reshape(1, NON_LOCAL, 1)
    pss_ref[...] = jnp.sum(acc * acc, axis=1).reshape(1, NON_LOCAL, 1)


def _conv(xmT, kmatT, conv_b):
    return pl.pallas_call(
        _conv_kernel,
        grid=(NT,),
        compiler_params=_PAR,
        in_specs=[_cm(HH), _cm_prev(HH), _cm_next(HH),
                  pl.BlockSpec((9, NON_LOCAL, HH), lambda m: (0, 0, 0)),
                  _full(NON_LOCAL, 1)],
        out_specs=[_cm(NON_LOCAL),
                   pl.BlockSpec((1, NON_LOCAL, 1), lambda m: (m, 0, 0)),
                   pl.BlockSpec((1, NON_LOCAL, 1), lambda m: (m, 0, 0))],
        out_shape=[jax.ShapeDtypeStruct((NON_LOCAL, R), jnp.bfloat16),
                   jax.ShapeDtypeStruct((NT, NON_LOCAL, 1), jnp.float32),
                   jax.ShapeDtypeStruct((NT, NON_LOCAL, 1), jnp.float32)],
    )(xmT, xmT, xmT, kmatT.astype(jnp.bfloat16), conv_b.reshape(NON_LOCAL, 1))


def _stats_kernel(ps_ref, pss_ref, g_ref, b_ref, scale_ref, shift_ref):
    mu = jnp.sum(ps_ref[...].reshape(NT, NON_LOCAL), axis=0) \
        .reshape(NON_LOCAL, 1) / R
    ex2 = jnp.sum(pss_ref[...].reshape(NT, NON_LOCAL), axis=0) \
        .reshape(NON_LOCAL, 1) / R
    var = ex2 - mu * mu
    scale = g_ref[...] * jax.lax.rsqrt(var + 1e-5)
    scale_ref[...] = scale
    shift_ref[...] = b_ref[...] - mu * scale


def _stats(ps, pss, bn_g, bn_b):
    return pl.pallas_call(
        _stats_kernel,
        out_shape=[jax.ShapeDtypeStruct((NON_LOCAL, 1), jnp.float32),
                   jax.ShapeDtypeStruct((NON_LOCAL, 1), jnp.float32)],
    )(ps, pss, bn_g.reshape(NON_LOCAL, 1), bn_b.reshape(NON_LOCAL, 1))


def _mlp_kernel(hx_ref, cv_ref, ps_ref, pss_ref, g_ref, bb_ref, cur_ref,
                w0a_ref, w0b_ref, b0_ref, w1_ref, b1_ref, w2_ref, b2_ref,
                out_ref):
    mu = jnp.sum(ps_ref[...], axis=0) / R
    var = jnp.sum(pss_ref[...], axis=0) / R - mu * mu
    scale = g_ref[...] * jax.lax.rsqrt(var + 1e-5)
    shift = bb_ref[...] - mu * scale
    nl = _lrelu(cv_ref[...].astype(jnp.float32) * scale + shift, 0.01)
    z = (jnp.dot(w0a_ref[...], hx_ref[...],
                 preferred_element_type=jnp.float32)
         + jnp.dot(w0b_ref[...], nl.astype(jnp.bfloat16),
                   preferred_element_type=jnp.float32) + b0_ref[...])
    z = _lrelu(z, 0.01).astype(jnp.bfloat16)
    z = _lrelu(jnp.dot(w1_ref[...], z, preferred_element_type=jnp.float32)
               + b1_ref[...], 0.01).astype(jnp.bfloat16)
    out_ref[...] = (jnp.dot(w2_ref[...], z, preferred_element_type=jnp.float32)
                    + b2_ref[...] + cur_ref[...])


def _mlp(hxT, convT, ps, pss, bn_g, bn_b, curT, w0, b0, w1, b1, w2, b2,
         target):
    return pl.pallas_call(
        _mlp_kernel,
        grid=(NT,),
        compiler_params=_PAR,
        in_specs=[_cm(HH), _cm(NON_LOCAL),
                  pl.BlockSpec((NT, NON_LOCAL, 1), lambda m: (0, 0, 0)),
                  pl.BlockSpec((NT, NON_LOCAL, 1), lambda m: (0, 0, 0)),
                  _full(NON_LOCAL, 1), _full(NON_LOCAL, 1), _cm(target),
                  _full(512, HH), _full(512, NON_LOCAL), _full(512, 1),
                  _full(256, 512), _full(256, 1),
                  _full(target, 256), _full(target, 1)],
        out_specs=_cm(target),
        out_shape=jax.ShapeDtypeStruct((target, R), jnp.float32),
    )(hxT, convT, ps, pss, bn_g.reshape(NON_LOCAL, 1),
      bn_b.reshape(NON_LOCAL, 1), curT,
      w0[:HH].T.astype(jnp.bfloat16), w0[HH:].T.astype(jnp.bfloat16),
      b0.reshape(512, 1), w1.T.astype(jnp.bfloat16), b1.reshape(256, 1),
      w2.T.astype(jnp.bfloat16), b2.reshape(target, 1))


def _att_mat(att):
    out = jnp.zeros((HEADS, HIDDEN, HEADS), jnp.float32)
    for hd in range(HEADS):
        out = out.at[hd, :, hd].set(att[hd])
    return out.reshape(HH, HEADS)


def _edge_mat(w_e, att_e):
    mm = (w_e.reshape(EDGE_DIM, HEADS, HIDDEN) * att_e[None]).sum(-1)
    out = jnp.zeros((4, EDGE_DIM, 4, HEADS), jnp.float32)
    for d in range(4):
        out = out.at[d, :, d, :].set(mm)
    return out.reshape(4 * EDGE_DIM, 4 * HEADS)


def _block(xpT, consT, e32T, p, target):
    h2 = _gat_block([xpT, consT], e32T, p['gat'])
    kmatT = jnp.transpose(p['conv_w'], (2, 3, 0, 1)) \
        .reshape(9, NON_LOCAL, HH)
    conv, ps, pss = _conv(h2, kmatT, p['conv_b'])
    return _mlp(h2, conv, ps, pss, p['bn_g'], p['bn_b'], xpT[-target:],
                p['mlp_w0'], p['mlp_b0'], p['mlp_w1'], p['mlp_b1'],
                p['mlp_w2'], p['mlp_b2'], target)


def kernel(x, x_cons, time_embedding, y, lat, edge_attr, edge_index, params,
           metric=None):
    te = -jnp.cos(2.0 * np.pi * time_embedding / 8760.0)
    te = jnp.broadcast_to(te.reshape(B, 1, 1), (B, 1, H * W))
    consT = jnp.concatenate([x_cons.reshape(B, N_CONST, H * W), te], axis=1)
    consT = jnp.transpose(consT, (1, 0, 2)).reshape(N_CONST + 1, R)
    xpT = jnp.transpose(x.reshape(B, INNER_DIM, H * W),
                        (1, 0, 2)).reshape(INNER_DIM, R)
    e32T = jnp.transpose(edge_attr.reshape(R, 4 * EDGE_DIM), (1, 0))

    out1 = _block(xpT, consT, e32T, params['blocks'][0], INNER_DIM)
    out2 = _block(out1, consT, e32T, params['blocks'][1], N_TRACED)
    return jnp.transpose(out2.reshape(N_TRACED, B, H * W),
                         (1, 0, 2)).reshape(B, N_TRACED, H, W)
